# Initial kernel scaffold; baseline (speedup 1.0000x reference)
import jax
import jax.numpy as jnp
from jax.experimental import pallas as pl


def kernel(feat_ids, feat_vals, LR_W, LR_B):
    raise NotImplementedError("write your pallas kernel here")


# speedup vs baseline: 1.8851x; 1.8851x over previous
import functools

import jax
import jax.numpy as jnp
from jax import lax
from jax.experimental import pallas as pl
from jax.experimental.pallas import tpu as pltpu
from jax.experimental.pallas import tpu_sc as plsc

FIELD = 26
BATCH = 16384
LANES = 16
NC = 2
NS = 16
NW = NC * NS
ROWS_W = BATCH // NW
CHUNKS = ROWS_W // LANES
FLAT = FIELD * ROWS_W


def _sc_body(ids_hbm, vals_hbm, w_hbm, b_hbm, out_hbm,
             idx_v, vals_v, g_v, out_v, b_v, sem):
    c = lax.axis_index("c")
    s = lax.axis_index("s")
    wid = s * NC + c

    pltpu.sync_copy(ids_hbm.at[wid], idx_v)
    pltpu.sync_copy(vals_hbm.at[wid], vals_v)
    pltpu.sync_copy(b_hbm, b_v)
    pltpu.async_copy(w_hbm.at[idx_v], g_v, sem).wait()

    bias = b_v[...]

    def chunk(ci, carry):
        base = ci * LANES
        acc = jnp.zeros((LANES,), jnp.float32)
        for j in range(FIELD):
            off = j * ROWS_W + base
            acc = acc + g_v[pl.ds(off, LANES)] * vals_v[pl.ds(off, LANES)]
        z = acc + bias
        out_v[pl.ds(base, LANES)] = 1.0 / (1.0 + jnp.exp(-z))
        return carry

    lax.fori_loop(0, CHUNKS, chunk, 0)
    pltpu.sync_copy(out_v, out_hbm.at[pl.ds(wid * ROWS_W, ROWS_W)])


_sc_kernel = functools.partial(
    pl.kernel,
    out_type=jax.ShapeDtypeStruct((BATCH,), jnp.float32),
    mesh=plsc.VectorSubcoreMesh(core_axis_name="c", subcore_axis_name="s"),
    scratch_types=[
        pltpu.VMEM((FLAT,), jnp.int32),
        pltpu.VMEM((FLAT,), jnp.float32),
        pltpu.VMEM((FLAT,), jnp.float32),
        pltpu.VMEM((ROWS_W,), jnp.float32),
        pltpu.VMEM((LANES,), jnp.float32),
        pltpu.SemaphoreType.DMA,
    ],
)(_sc_body)


def kernel(feat_ids, feat_vals, LR_W, LR_B):
    ids_t = feat_ids.reshape(NW, ROWS_W, FIELD).transpose(0, 2, 1).reshape(NW, FLAT)
    vals_t = feat_vals.reshape(NW, ROWS_W, FIELD).transpose(0, 2, 1).reshape(NW, FLAT)
    b16 = jnp.broadcast_to(LR_B, (LANES,))
    return _sc_kernel(ids_t, vals_t, LR_W, b16)
